# baseline (speedup 1.0000x reference)
import functools
import math

import jax
import jax.numpy as jnp
from jax import lax
from jax.experimental import pallas as pl
from jax.experimental.pallas import tpu as pltpu
from jax.experimental.pallas import tpu_sc as plsc

D_MODEL = 256
NUM_POS_FEATS = D_MODEL // 2
TEMPERATURE = 10000.0
ROW_BLOCK = 16384

_C1 = 6.278553964015136
_C3 = -41.09111633904149
_C5 = 77.90940338850729
_C7 = -56.03846993503516


def _sin2pi(t):
    u = t - jnp.round(t)
    u2 = u * u
    p = _C7
    p = p * u2 + _C5
    p = p * u2 + _C3
    p = p * u2 + _C1
    return p * u


def _emb_body(pts_ref, out_ref):
    pts = pts_ref[...]
    x = pts[:, 0:1]
    y = pts[:, 1:2]

    lane = jax.lax.broadcasted_iota(jnp.int32, (1, NUM_POS_FEATS), 1)
    expo = (lane // 2).astype(jnp.float32) * (2.0 / NUM_POS_FEATS)
    inv_dim_t = jnp.exp(expo * (-math.log(TEMPERATURE)))
    phase = (lane % 2).astype(jnp.float32) * 0.25

    out_ref[:, :NUM_POS_FEATS] = _sin2pi(y * inv_dim_t + phase)
    out_ref[:, NUM_POS_FEATS:] = _sin2pi(x * inv_dim_t + phase)


def _tc_emb(flat, n):
    return pl.pallas_call(
        _emb_body,
        grid=(n // ROW_BLOCK,),
        in_specs=[pl.BlockSpec((ROW_BLOCK, 2), lambda i: (i, 0))],
        out_specs=pl.BlockSpec((ROW_BLOCK, D_MODEL), lambda i: (i, 0)),
        out_shape=jax.ShapeDtypeStruct((n, D_MODEL), jnp.float32),
        compiler_params=pltpu.CompilerParams(
            dimension_semantics=("parallel",)),
    )(flat)


_MAGIC = 1.5 * 2.0 ** 23
_CH = 128


def _sc_poly(t):
    r = (t + _MAGIC) - _MAGIC
    u = t - r
    u2 = u * u
    p = _C7
    p = p * u2 + _C5
    p = p * u2 + _C3
    p = p * u2 + _C1
    return p * u


def _sc_body(xs_hbm, ys_hbm, out_hbm, in_vx, in_vy, out_v0, out_v1, sem0, sem1, n_sc):
    info = plsc.get_sparse_core_info()
    nc, ns = info.num_cores, info.num_subcores
    nw = nc * ns
    wid = lax.axis_index("s") * nc + lax.axis_index("c")
    rows_w = n_sc // nw
    base = wid * rows_w
    n_chunks = rows_w // _CH

    lane16 = lax.iota(jnp.int32, 16)
    c_inv, c_ph = [], []
    for g in range(8):
        jj = lane16 + (16 * g)
        kk = jnp.float32(2.0 / NUM_POS_FEATS) * (jj >> 1).astype(jnp.float32)
        c_inv.append(jnp.exp(kk * jnp.float32(-math.log(TEMPERATURE))))
        c_ph.append((jj & 1).astype(jnp.float32) * jnp.float32(0.25))

    splat = [jnp.full((16,), e, jnp.int32) for e in range(16)]

    def do_chunk(c, out_v, sem):
        row0 = base + c * _CH
        pltpu.sync_copy(xs_hbm.at[pl.ds(row0, _CH)], in_vx)
        pltpu.sync_copy(ys_hbm.at[pl.ds(row0, _CH)], in_vy)

        def block16(b, carry):
            xb = in_vx[pl.ds(16 * b, 16)]
            yb = in_vy[pl.ds(16 * b, 16)]
            for e in range(16):
                i = 16 * b + e
                xv = xb.at[splat[e]].get(mode="promise_in_bounds")
                yv = yb.at[splat[e]].get(mode="promise_in_bounds")
                for g in range(8):
                    out_v[i, pl.ds(16 * g, 16)] = _sc_poly(
                        yv * c_inv[g] + c_ph[g])
                    out_v[i, pl.ds(128 + 16 * g, 16)] = _sc_poly(
                        xv * c_inv[g] + c_ph[g])
            return carry

        lax.fori_loop(0, _CH // 16, block16, 0)
        pltpu.async_copy(out_v, out_hbm.at[pl.ds(row0, _CH)], sem)

    def wait_prev(out_v, sem):
        pltpu.make_async_copy(out_v, out_hbm.at[pl.ds(0, _CH)], sem).wait()

    def pair(c2, carry):
        c = 2 * c2

        @pl.when(c2 > 0)
        def _():
            wait_prev(out_v0, sem0)

        do_chunk(c, out_v0, sem0)

        @pl.when(c2 > 0)
        def _():
            wait_prev(out_v1, sem1)

        do_chunk(c + 1, out_v1, sem1)
        return carry

    lax.fori_loop(0, n_chunks // 2, pair, 0)
    wait_prev(out_v0, sem0)
    wait_prev(out_v1, sem1)


def _sc_emb(flat, n_sc):
    mesh = plsc.VectorSubcoreMesh(core_axis_name="c", subcore_axis_name="s")
    k = functools.partial(
        pl.kernel,
        mesh=mesh,
        out_type=jax.ShapeDtypeStruct((n_sc, D_MODEL), jnp.float32),
        scratch_types=[
            pltpu.VMEM((_CH,), jnp.float32),
            pltpu.VMEM((_CH,), jnp.float32),
            pltpu.VMEM((_CH, D_MODEL), jnp.float32),
            pltpu.VMEM((_CH, D_MODEL), jnp.float32),
            pltpu.SemaphoreType.DMA,
            pltpu.SemaphoreType.DMA,
        ],
    )(functools.partial(_sc_body, n_sc=n_sc))
    return k(flat[:, 0], flat[:, 1])


N_SC = 40960


def kernel(points):
    b, p, _ = points.shape
    n = b * p
    flat = points.reshape(n, 2)
    n_tc = n - N_SC
    tc_out = _tc_emb(flat[:n_tc], n_tc)
    sc_out = _sc_emb(flat[n_tc:], N_SC)
    out = jnp.concatenate([tc_out, sc_out], axis=0)
    return out.reshape(b, p, D_MODEL)


# speedup vs baseline: 2.3779x; 2.3779x over previous
import functools
import math

import jax
import jax.numpy as jnp
from jax import lax
from jax.experimental import pallas as pl
from jax.experimental.pallas import tpu as pltpu
from jax.experimental.pallas import tpu_sc as plsc

D_MODEL = 256
NUM_POS_FEATS = D_MODEL // 2
TEMPERATURE = 10000.0
ROW_BLOCK = 16384

_C1 = 6.278553964015136
_C3 = -41.09111633904149
_C5 = 77.90940338850729
_C7 = -56.03846993503516


def _sin2pi(t):
    u = t - jnp.round(t)
    u2 = u * u
    p = _C7
    p = p * u2 + _C5
    p = p * u2 + _C3
    p = p * u2 + _C1
    return p * u


def _emb_body(pts_ref, out_ref):
    pts = pts_ref[...]
    x = pts[:, 0:1]
    y = pts[:, 1:2]

    lane = jax.lax.broadcasted_iota(jnp.int32, (1, NUM_POS_FEATS), 1)
    expo = (lane // 2).astype(jnp.float32) * (2.0 / NUM_POS_FEATS)
    inv_dim_t = jnp.exp(expo * (-math.log(TEMPERATURE)))
    phase = (lane % 2).astype(jnp.float32) * 0.25

    out_ref[:, :NUM_POS_FEATS] = _sin2pi(y * inv_dim_t + phase)
    out_ref[:, NUM_POS_FEATS:] = _sin2pi(x * inv_dim_t + phase)


def _tc_emb(flat, n):
    return pl.pallas_call(
        _emb_body,
        grid=((n + ROW_BLOCK - 1) // ROW_BLOCK,),
        in_specs=[pl.BlockSpec((ROW_BLOCK, 2), lambda i: (i, 0))],
        out_specs=pl.BlockSpec((ROW_BLOCK, D_MODEL), lambda i: (i, 0)),
        out_shape=jax.ShapeDtypeStruct((n, D_MODEL), jnp.float32),
        compiler_params=pltpu.CompilerParams(
            dimension_semantics=("parallel",)),
    )(flat)


_MAGIC = 1.5 * 2.0 ** 23
_CH = 128


def _sc_poly(t):
    r = (t + _MAGIC) - _MAGIC
    u = t - r
    u2 = u * u
    p = _C7
    p = p * u2 + _C5
    p = p * u2 + _C3
    p = p * u2 + _C1
    return p * u


def _sc_body(xs_hbm, ys_hbm, out_hbm, in_vx, in_vy, out_v0, out_v1, sem0, sem1, n_sc):
    info = plsc.get_sparse_core_info()
    nc, ns = info.num_cores, info.num_subcores
    nw = nc * ns
    wid = lax.axis_index("s") * nc + lax.axis_index("c")
    rows_w = n_sc // nw
    base = wid * rows_w
    n_chunks = rows_w // _CH

    lane16 = lax.iota(jnp.int32, 16)
    c_inv, c_ph = [], []
    for g in range(8):
        jj = lane16 + (16 * g)
        kk = jnp.float32(2.0 / NUM_POS_FEATS) * (jj >> 1).astype(jnp.float32)
        c_inv.append(jnp.exp(kk * jnp.float32(-math.log(TEMPERATURE))))
        c_ph.append((jj & 1).astype(jnp.float32) * jnp.float32(0.25))

    splat = [jnp.full((16,), e, jnp.int32) for e in range(16)]

    def do_chunk(c, out_v, sem):
        row0 = base + c * _CH
        pltpu.sync_copy(xs_hbm.at[pl.ds(row0, _CH)], in_vx)
        pltpu.sync_copy(ys_hbm.at[pl.ds(row0, _CH)], in_vy)

        def block16(b, carry):
            xb = in_vx[pl.ds(16 * b, 16)]
            yb = in_vy[pl.ds(16 * b, 16)]
            for e in range(16):
                i = 16 * b + e
                xv = xb.at[splat[e]].get(mode="promise_in_bounds")
                yv = yb.at[splat[e]].get(mode="promise_in_bounds")
                for g in range(8):
                    out_v[i, pl.ds(16 * g, 16)] = _sc_poly(
                        yv * c_inv[g] + c_ph[g])
                    out_v[i, pl.ds(128 + 16 * g, 16)] = _sc_poly(
                        xv * c_inv[g] + c_ph[g])
            return carry

        lax.fori_loop(0, _CH // 16, block16, 0)
        pltpu.async_copy(out_v, out_hbm.at[pl.ds(row0, _CH)], sem)

    def wait_prev(out_v, sem):
        pltpu.make_async_copy(out_v, out_hbm.at[pl.ds(0, _CH)], sem).wait()

    def pair(c2, carry):
        c = 2 * c2

        @pl.when(c2 > 0)
        def _():
            wait_prev(out_v0, sem0)

        do_chunk(c, out_v0, sem0)

        @pl.when(c2 > 0)
        def _():
            wait_prev(out_v1, sem1)

        do_chunk(c + 1, out_v1, sem1)
        return carry

    lax.fori_loop(0, n_chunks // 2, pair, 0)
    wait_prev(out_v0, sem0)
    wait_prev(out_v1, sem1)


def _sc_emb(flat, n_sc):
    mesh = plsc.VectorSubcoreMesh(core_axis_name="c", subcore_axis_name="s")
    k = functools.partial(
        pl.kernel,
        mesh=mesh,
        out_type=jax.ShapeDtypeStruct((n_sc, D_MODEL), jnp.float32),
        scratch_types=[
            pltpu.VMEM((_CH,), jnp.float32),
            pltpu.VMEM((_CH,), jnp.float32),
            pltpu.VMEM((_CH, D_MODEL), jnp.float32),
            pltpu.VMEM((_CH, D_MODEL), jnp.float32),
            pltpu.SemaphoreType.DMA,
            pltpu.SemaphoreType.DMA,
        ],
    )(functools.partial(_sc_body, n_sc=n_sc))
    return k(flat[:, 0], flat[:, 1])


N_SC = 40960


def kernel(points):
    b, p, _ = points.shape
    n = b * p
    flat = points.reshape(n, 2)
    out = _tc_emb(flat, n)
    return out.reshape(b, p, D_MODEL)


# speedup vs baseline: 2.3823x; 1.0019x over previous
import functools
import math

import jax
import jax.numpy as jnp
from jax import lax
from jax.experimental import pallas as pl
from jax.experimental.pallas import tpu as pltpu
from jax.experimental.pallas import tpu_sc as plsc

D_MODEL = 256
NUM_POS_FEATS = D_MODEL // 2
TEMPERATURE = 10000.0
ROW_BLOCK = 12800

_C1 = 6.278553964015136
_C3 = -41.09111633904149
_C5 = 77.90940338850729
_C7 = -56.03846993503516


def _sin2pi(t):
    u = t - jnp.round(t)
    u2 = u * u
    p = _C7
    p = p * u2 + _C5
    p = p * u2 + _C3
    p = p * u2 + _C1
    return p * u


def _emb_body(pts_ref, out_ref):
    pts = pts_ref[...]
    x = pts[:, 0:1]
    y = pts[:, 1:2]

    lane = jax.lax.broadcasted_iota(jnp.int32, (1, NUM_POS_FEATS), 1)
    expo = (lane // 2).astype(jnp.float32) * (2.0 / NUM_POS_FEATS)
    inv_dim_t = jnp.exp(expo * (-math.log(TEMPERATURE)))
    phase = (lane % 2).astype(jnp.float32) * 0.25

    out_ref[:, :NUM_POS_FEATS] = _sin2pi(y * inv_dim_t + phase)
    out_ref[:, NUM_POS_FEATS:] = _sin2pi(x * inv_dim_t + phase)


def _tc_emb(flat, n):
    return pl.pallas_call(
        _emb_body,
        grid=(n // ROW_BLOCK,),
        in_specs=[pl.BlockSpec((ROW_BLOCK, 2), lambda i: (i, 0))],
        out_specs=pl.BlockSpec((ROW_BLOCK, D_MODEL), lambda i: (i, 0)),
        out_shape=jax.ShapeDtypeStruct((n, D_MODEL), jnp.float32),
        compiler_params=pltpu.CompilerParams(
            dimension_semantics=("parallel",)),
    )(flat)


_MAGIC = 1.5 * 2.0 ** 23
_CH = 128


def _sc_poly(t):
    r = (t + _MAGIC) - _MAGIC
    u = t - r
    u2 = u * u
    p = _C7
    p = p * u2 + _C5
    p = p * u2 + _C3
    p = p * u2 + _C1
    return p * u


def _sc_body(xs_hbm, ys_hbm, out_hbm, in_vx, in_vy, out_v0, out_v1, sem0, sem1, n_sc):
    info = plsc.get_sparse_core_info()
    nc, ns = info.num_cores, info.num_subcores
    nw = nc * ns
    wid = lax.axis_index("s") * nc + lax.axis_index("c")
    rows_w = n_sc // nw
    base = wid * rows_w
    n_chunks = rows_w // _CH

    lane16 = lax.iota(jnp.int32, 16)
    c_inv, c_ph = [], []
    for g in range(8):
        jj = lane16 + (16 * g)
        kk = jnp.float32(2.0 / NUM_POS_FEATS) * (jj >> 1).astype(jnp.float32)
        c_inv.append(jnp.exp(kk * jnp.float32(-math.log(TEMPERATURE))))
        c_ph.append((jj & 1).astype(jnp.float32) * jnp.float32(0.25))

    splat = [jnp.full((16,), e, jnp.int32) for e in range(16)]

    def do_chunk(c, out_v, sem):
        row0 = base + c * _CH
        pltpu.sync_copy(xs_hbm.at[pl.ds(row0, _CH)], in_vx)
        pltpu.sync_copy(ys_hbm.at[pl.ds(row0, _CH)], in_vy)

        def block16(b, carry):
            xb = in_vx[pl.ds(16 * b, 16)]
            yb = in_vy[pl.ds(16 * b, 16)]
            for e in range(16):
                i = 16 * b + e
                xv = xb.at[splat[e]].get(mode="promise_in_bounds")
                yv = yb.at[splat[e]].get(mode="promise_in_bounds")
                for g in range(8):
                    out_v[i, pl.ds(16 * g, 16)] = _sc_poly(
                        yv * c_inv[g] + c_ph[g])
                    out_v[i, pl.ds(128 + 16 * g, 16)] = _sc_poly(
                        xv * c_inv[g] + c_ph[g])
            return carry

        lax.fori_loop(0, _CH // 16, block16, 0)
        pltpu.async_copy(out_v, out_hbm.at[pl.ds(row0, _CH)], sem)

    def wait_prev(out_v, sem):
        pltpu.make_async_copy(out_v, out_hbm.at[pl.ds(0, _CH)], sem).wait()

    def pair(c2, carry):
        c = 2 * c2

        @pl.when(c2 > 0)
        def _():
            wait_prev(out_v0, sem0)

        do_chunk(c, out_v0, sem0)

        @pl.when(c2 > 0)
        def _():
            wait_prev(out_v1, sem1)

        do_chunk(c + 1, out_v1, sem1)
        return carry

    lax.fori_loop(0, n_chunks // 2, pair, 0)
    wait_prev(out_v0, sem0)
    wait_prev(out_v1, sem1)


def _sc_emb(flat, n_sc):
    mesh = plsc.VectorSubcoreMesh(core_axis_name="c", subcore_axis_name="s")
    k = functools.partial(
        pl.kernel,
        mesh=mesh,
        out_type=jax.ShapeDtypeStruct((n_sc, D_MODEL), jnp.float32),
        scratch_types=[
            pltpu.VMEM((_CH,), jnp.float32),
            pltpu.VMEM((_CH,), jnp.float32),
            pltpu.VMEM((_CH, D_MODEL), jnp.float32),
            pltpu.VMEM((_CH, D_MODEL), jnp.float32),
            pltpu.SemaphoreType.DMA,
            pltpu.SemaphoreType.DMA,
        ],
    )(functools.partial(_sc_body, n_sc=n_sc))
    return k(flat[:, 0], flat[:, 1])


N_SC = 40960


def kernel(points):
    b, p, _ = points.shape
    n = b * p
    flat = points.reshape(n, 2)
    out = _tc_emb(flat, n)
    return out.reshape(b, p, D_MODEL)
